# baseline (speedup 1.0000x reference)
import functools

import jax
import jax.numpy as jnp
from jax import lax
from jax.experimental import pallas as pl
from jax.experimental.pallas import tpu as pltpu
from jax.experimental.pallas import tpu_sc as plsc


def _gather_rows(N, D, NC, NS, chunk):
    NW = NC * NS
    per_w = N // NW
    n_chunks = per_w // chunk
    mesh = plsc.VectorSubcoreMesh(core_axis_name="c", subcore_axis_name="s")

    @functools.partial(
        pl.kernel,
        mesh=mesh,
        out_type=jax.ShapeDtypeStruct((N, D), jnp.float32),
        scratch_types=[
            pltpu.VMEM((chunk,), jnp.int32),
            pltpu.VMEM((chunk, D), jnp.float32),
            pltpu.SemaphoreType.DMA,
        ],
        compiler_params=pltpu.CompilerParams(use_tc_tiling_on_sc=False),
    )
    def k(idx_hbm, table_hbm, out_hbm, idx_v, rows_v, sem):
        wid = lax.axis_index("s") * NC + lax.axis_index("c")
        base = wid * per_w

        def body(i, carry):
            off = base + i * chunk
            pltpu.sync_copy(idx_hbm.at[pl.ds(off, chunk)], idx_v)
            pltpu.async_copy(table_hbm.at[idx_v], rows_v, sem).wait()
            pltpu.sync_copy(rows_v, out_hbm.at[pl.ds(off, chunk)])
            return carry

        lax.fori_loop(0, n_chunks, body, 0)

    return k


def kernel(src, mask, emb_table):
    B, L = src.shape
    D = emb_table.shape[1]
    N = B * L
    info = plsc.get_sparse_core_info()
    idx_flat = src.reshape(N)
    out = _gather_rows(N, D, info.num_cores, info.num_subcores, 128)(
        idx_flat, emb_table
    )
    return out.reshape(B, L, D)


# speedup vs baseline: 1.1702x; 1.1702x over previous
import functools

import jax
import jax.numpy as jnp
from jax import lax
from jax.experimental import pallas as pl
from jax.experimental.pallas import tpu as pltpu
from jax.experimental.pallas import tpu_sc as plsc

_IB = 128


def _gather_rows(N, D, NC, NS, chunk):
    NW = NC * NS
    per_w = N // NW
    n_chunks = per_w // chunk
    n_sub = chunk // _IB
    assert n_chunks * chunk == per_w and n_sub * _IB == chunk
    mesh = plsc.VectorSubcoreMesh(core_axis_name="c", subcore_axis_name="s")

    @functools.partial(
        pl.kernel,
        mesh=mesh,
        out_type=jax.ShapeDtypeStruct((N, D), jnp.float32),
        scratch_types=[
            pltpu.VMEM((per_w // _IB, _IB), jnp.int32),
            pltpu.VMEM((chunk, D), jnp.float32),
            pltpu.SemaphoreType.DMA,
        ],
        compiler_params=pltpu.CompilerParams(use_tc_tiling_on_sc=False),
    )
    def k(idx_hbm, table_hbm, out_hbm, idx_v, rows_v, gsem):
        wid = lax.axis_index("s") * NC + lax.axis_index("c")
        base = wid * per_w
        pltpu.sync_copy(idx_hbm.at[wid], idx_v)

        def body(i, carry):
            for j in range(n_sub):
                pltpu.async_copy(
                    table_hbm.at[idx_v.at[i * n_sub + j]],
                    rows_v.at[pl.ds(j * _IB, _IB)],
                    gsem,
                )
            for j in range(n_sub):
                pltpu.make_async_copy(
                    table_hbm.at[idx_v.at[i * n_sub + j]],
                    rows_v.at[pl.ds(j * _IB, _IB)],
                    gsem,
                ).wait()
            pltpu.sync_copy(rows_v, out_hbm.at[pl.ds(base + i * chunk, chunk)])
            return carry

        lax.fori_loop(0, n_chunks, body, 0)

    return k


def kernel(src, mask, emb_table):
    B, L = src.shape
    D = emb_table.shape[1]
    N = B * L
    info = plsc.get_sparse_core_info()
    NW = info.num_cores * info.num_subcores
    idx = src.reshape(NW, (N // NW) // _IB, _IB)
    out = _gather_rows(N, D, info.num_cores, info.num_subcores, 512)(idx, emb_table)
    return out.reshape(B, L, D)


# speedup vs baseline: 1.1957x; 1.0218x over previous
import functools

import jax
import jax.numpy as jnp
from jax import lax
from jax.experimental import pallas as pl
from jax.experimental.pallas import tpu as pltpu
from jax.experimental.pallas import tpu_sc as plsc

_IB = 128


def _gather_rows(N, D, NC, NS, chunk):
    NW = NC * NS
    per_w = N // NW
    n_chunks = per_w // chunk
    n_sub = chunk // _IB
    assert n_chunks * chunk == per_w and n_sub * _IB == chunk
    assert n_chunks % 2 == 0 and n_chunks >= 4
    mesh = plsc.VectorSubcoreMesh(core_axis_name="c", subcore_axis_name="s")

    @functools.partial(
        pl.kernel,
        mesh=mesh,
        out_type=jax.ShapeDtypeStruct((N, D), jnp.float32),
        scratch_types=[
            pltpu.VMEM((per_w // _IB, _IB), jnp.int32),
            pltpu.VMEM((2, chunk, D), jnp.float32),
            pltpu.SemaphoreType.DMA,
            pltpu.SemaphoreType.DMA,
            pltpu.SemaphoreType.DMA,
            pltpu.SemaphoreType.DMA,
        ],
        compiler_params=pltpu.CompilerParams(use_tc_tiling_on_sc=False),
    )
    def k(idx_hbm, table_hbm, out_hbm, idx_v, rows_v, g0, g1, w0, w1):
        wid = lax.axis_index("s") * NC + lax.axis_index("c")
        base = wid * per_w
        gsem = (g0, g1)
        wsem = (w0, w1)
        pltpu.sync_copy(idx_hbm.at[wid], idx_v)

        def gather_descs(i, b):
            for j in range(n_sub):
                yield (
                    table_hbm.at[idx_v.at[i * n_sub + j]],
                    rows_v.at[b].at[pl.ds(j * _IB, _IB)],
                    gsem[b],
                )

        def start_gather(i, b):
            for args in gather_descs(i, b):
                pltpu.async_copy(*args)

        def wait_gather(i, b):
            for args in gather_descs(i, b):
                pltpu.make_async_copy(*args).wait()

        def write_desc(i, b):
            return (
                rows_v.at[b],
                out_hbm.at[pl.ds(base + i * chunk, chunk)],
                wsem[b],
            )

        def start_write(i, b):
            pltpu.async_copy(*write_desc(i, b))

        def wait_write(i, b):
            pltpu.make_async_copy(*write_desc(i, b)).wait()

        start_gather(0, 0)
        wait_gather(0, 0)
        start_write(0, 0)
        start_gather(1, 1)

        def body(g, carry):
            i1 = 2 * g + 1
            wait_gather(i1, 1)
            start_write(i1, 1)
            wait_write(i1 - 1, 0)
            start_gather(i1 + 1, 0)
            i2 = 2 * g + 2
            wait_gather(i2, 0)
            start_write(i2, 0)
            wait_write(i2 - 1, 1)
            start_gather(i2 + 1, 1)
            return carry

        lax.fori_loop(0, n_chunks // 2 - 1, body, 0)

        last = n_chunks - 1
        wait_gather(last, 1)
        start_write(last, 1)
        wait_write(last - 1, 0)
        wait_write(last, 1)

    return k


def kernel(src, mask, emb_table):
    B, L = src.shape
    D = emb_table.shape[1]
    N = B * L
    info = plsc.get_sparse_core_info()
    NW = info.num_cores * info.num_subcores
    idx = src.reshape(NW, (N // NW) // _IB, _IB)
    out = _gather_rows(N, D, info.num_cores, info.num_subcores, 512)(idx, emb_table)
    return out.reshape(B, L, D)


# speedup vs baseline: 1.1979x; 1.0018x over previous
import functools

import jax
import jax.numpy as jnp
from jax import lax
from jax.experimental import pallas as pl
from jax.experimental.pallas import tpu as pltpu
from jax.experimental.pallas import tpu_sc as plsc

_IB = 80
_RPC = 2


def _gather_rows(B, L, D, NC, NS):
    NW = NC * NS
    b_per_w = B // NW
    per_w = b_per_w * L
    chunk = _RPC * L
    n_sub = chunk // _IB
    n_chunks = per_w // chunk
    assert n_sub * _IB == chunk and n_chunks * chunk == per_w
    assert n_chunks % 2 == 0 and n_chunks >= 4
    mesh = plsc.VectorSubcoreMesh(core_axis_name="c", subcore_axis_name="s")

    @functools.partial(
        pl.kernel,
        mesh=mesh,
        out_type=jax.ShapeDtypeStruct((B, L, D), jnp.float32),
        scratch_types=[
            pltpu.VMEM((per_w // _IB, _IB), jnp.int32),
            pltpu.VMEM((2, chunk, D), jnp.float32),
            pltpu.SemaphoreType.DMA,
            pltpu.SemaphoreType.DMA,
            pltpu.SemaphoreType.DMA,
            pltpu.SemaphoreType.DMA,
        ],
        compiler_params=pltpu.CompilerParams(use_tc_tiling_on_sc=False),
    )
    def k(idx_hbm, table_hbm, out_hbm, idx_v, rows_v, g0, g1, w0, w1):
        wid = lax.axis_index("s") * NC + lax.axis_index("c")
        b_base = wid * b_per_w
        gsem = (g0, g1)
        wsem = (w0, w1)
        pltpu.sync_copy(idx_hbm.at[wid], idx_v)

        def gather_descs(i, b):
            for j in range(n_sub):
                yield (
                    table_hbm.at[idx_v.at[i * n_sub + j]],
                    rows_v.at[b].at[pl.ds(j * _IB, _IB)],
                    gsem[b],
                )

        def start_gather(i, b):
            for args in gather_descs(i, b):
                pltpu.async_copy(*args)

        def wait_gather(i, b):
            for args in gather_descs(i, b):
                pltpu.make_async_copy(*args).wait()

        def write_descs(i, b):
            for r in range(_RPC):
                yield (
                    rows_v.at[b].at[pl.ds(r * L, L)],
                    out_hbm.at[b_base + i * _RPC + r],
                    wsem[b],
                )

        def start_write(i, b):
            for args in write_descs(i, b):
                pltpu.async_copy(*args)

        def wait_write(i, b):
            for args in write_descs(i, b):
                pltpu.make_async_copy(*args).wait()

        start_gather(0, 0)
        wait_gather(0, 0)
        start_write(0, 0)
        start_gather(1, 1)

        def body(g, carry):
            i1 = 2 * g + 1
            wait_gather(i1, 1)
            start_write(i1, 1)
            wait_write(i1 - 1, 0)
            start_gather(i1 + 1, 0)
            i2 = 2 * g + 2
            wait_gather(i2, 0)
            start_write(i2, 0)
            wait_write(i2 - 1, 1)
            start_gather(i2 + 1, 1)
            return carry

        lax.fori_loop(0, n_chunks // 2 - 1, body, 0)

        last = n_chunks - 1
        wait_gather(last, 1)
        start_write(last, 1)
        wait_write(last - 1, 0)
        wait_write(last, 1)

    return k


def kernel(src, mask, emb_table):
    B, L = src.shape
    D = emb_table.shape[1]
    info = plsc.get_sparse_core_info()
    NW = info.num_cores * info.num_subcores
    idx = src.reshape(NW, (B * L // NW) // _IB, _IB)
    return _gather_rows(B, L, D, info.num_cores, info.num_subcores)(idx, emb_table)


# speedup vs baseline: 1.5888x; 1.3264x over previous
import functools

import jax
import jax.numpy as jnp
from jax import lax
from jax.experimental import pallas as pl
from jax.experimental.pallas import tpu as pltpu
from jax.experimental.pallas import tpu_sc as plsc

_IB = 80
_RPC = 2


def _gather_rows(B, L, D, NC, NS):
    NW = NC * NS
    b_per_w = B // NW
    per_w = b_per_w * L
    chunk = _RPC * L
    n_sub = chunk // _IB
    n_chunks = per_w // chunk
    assert n_sub * _IB == chunk and n_chunks * chunk == per_w
    assert n_chunks % 2 == 0 and n_chunks >= 4
    mesh = plsc.VectorSubcoreMesh(core_axis_name="c", subcore_axis_name="s")

    @functools.partial(
        pl.kernel,
        mesh=mesh,
        out_type=jax.ShapeDtypeStruct((B, L, 2 * D), jnp.float32),
        scratch_types=[
            pltpu.VMEM((per_w // _IB, _IB), jnp.int32),
            pltpu.VMEM((2, chunk, D), jnp.float32),
            pltpu.SemaphoreType.DMA,
            pltpu.SemaphoreType.DMA,
            pltpu.SemaphoreType.DMA,
            pltpu.SemaphoreType.DMA,
        ],
        compiler_params=pltpu.CompilerParams(use_tc_tiling_on_sc=False),
    )
    def k(idx_hbm, table_hbm, out_hbm, idx_v, rows_v, g0, g1, w0, w1):
        wid = lax.axis_index("s") * NC + lax.axis_index("c")
        b_base = wid * b_per_w
        gsem = (g0, g1)
        wsem = (w0, w1)
        pltpu.sync_copy(idx_hbm.at[wid], idx_v)

        def gather_descs(i, b):
            for j in range(n_sub):
                yield (
                    table_hbm.at[idx_v.at[i * n_sub + j]],
                    rows_v.at[b].at[pl.ds(j * _IB, _IB)],
                    gsem[b],
                )

        def start_gather(i, b):
            for args in gather_descs(i, b):
                pltpu.async_copy(*args)

        def wait_gather(i, b):
            for args in gather_descs(i, b):
                pltpu.make_async_copy(*args).wait()

        def write_descs(i, b):
            for r in range(_RPC):
                yield (
                    rows_v.at[b].at[pl.ds(r * L, L)],
                    out_hbm.at[b_base + i * _RPC + r].at[:, pl.ds(0, D)],
                    wsem[b],
                )

        def start_write(i, b):
            for args in write_descs(i, b):
                pltpu.async_copy(*args)

        def wait_write(i, b):
            for args in write_descs(i, b):
                pltpu.make_async_copy(*args).wait()

        start_gather(0, 0)
        wait_gather(0, 0)
        start_write(0, 0)
        start_gather(1, 1)

        def body(g, carry):
            i1 = 2 * g + 1
            wait_gather(i1, 1)
            start_write(i1, 1)
            wait_write(i1 - 1, 0)
            start_gather(i1 + 1, 0)
            i2 = 2 * g + 2
            wait_gather(i2, 0)
            start_write(i2, 0)
            wait_write(i2 - 1, 1)
            start_gather(i2 + 1, 1)
            return carry

        lax.fori_loop(0, n_chunks // 2 - 1, body, 0)

        last = n_chunks - 1
        wait_gather(last, 1)
        start_write(last, 1)
        wait_write(last - 1, 0)
        wait_write(last, 1)

    return k


def kernel(src, mask, emb_table):
    B, L = src.shape
    D = emb_table.shape[1]
    info = plsc.get_sparse_core_info()
    NW = info.num_cores * info.num_subcores
    idx = src.reshape(NW, (B * L // NW) // _IB, _IB)
    out = _gather_rows(B, L, D, info.num_cores, info.num_subcores)(idx, emb_table)
    return out[:, :, :D]
